# Initial kernel scaffold; baseline (speedup 1.0000x reference)
import jax
import jax.numpy as jnp
from jax.experimental import pallas as pl


def kernel(x, edge_index, edge_attr, batch, lin0_w, lin0_b, nn1_w, nn1_b, nn2_w, nn2_b, conv_root, conv_bias, gru_w_ih, gru_w_hh, gru_b_ih, gru_b_hh, lstm_w_ih, lstm_w_hh, lstm_b_ih, lstm_b_hh, lin1_w, lin1_b):
    raise NotImplementedError("write your pallas kernel here")


# speedup vs baseline: 1.6940x; 1.6940x over previous
import functools

import jax
import jax.numpy as jnp
from jax import lax
from jax.experimental import pallas as pl
from jax.experimental.pallas import tpu as pltpu
from jax.experimental.pallas import tpu_sc as plsc

_N = 10000
_E = 20000
_B = 512
_DIM = 64
_NF = 11
_DE = 5

_NPAD = 10240
_EPAD = 20480
_NC = 2
_NS = 16
_NW = _NC * _NS
_EPW = _EPAD // _NW
_JCH = _EPW // 128
_RPS = _NPAD // _NS


@functools.cache
def _get_sc_gather():
    mesh = plsc.VectorSubcoreMesh(core_axis_name="c", subcore_axis_name="s")

    @functools.partial(
        pl.kernel,
        out_type=jax.ShapeDtypeStruct((_NW, _EPW, _DIM), jnp.float32),
        mesh=mesh,
        scratch_types=[
            pltpu.VMEM((_JCH, 128), jnp.int32),
            pltpu.VMEM((_EPW, _DIM), jnp.float32),
            pltpu.SemaphoreType.DMA,
        ],
        compiler_params=pltpu.CompilerParams(use_tc_tiling_on_sc=False),
    )
    def _sc_gather_k(table_hbm, idx_hbm, out_hbm, idx_v, rows_v, sem):
        c = lax.axis_index("c")
        s = lax.axis_index("s")
        wid = s * _NC + c
        pltpu.sync_copy(idx_hbm.at[wid], idx_v)
        descs = []
        for j in range(_JCH):
            descs.append(
                pltpu.async_copy(
                    table_hbm.at[idx_v.at[j]], rows_v.at[pl.ds(j * 128, 128)],
                    sem,
                )
            )
        for d in descs:
            d.wait()
        pltpu.sync_copy(rows_v, out_hbm.at[wid])

    return _sc_gather_k


def _sc_gather(table, idx_r):
    return _get_sc_gather()(table, idx_r)


@functools.cache
def _get_sc_scatter(width):
    mesh = plsc.VectorSubcoreMesh(core_axis_name="c", subcore_axis_name="s")

    @functools.partial(
        pl.kernel,
        out_type=jax.ShapeDtypeStruct((_NC, _NS, _RPS, width), jnp.float32),
        mesh=mesh,
        scratch_types=[
            pltpu.VMEM((_JCH, 128), jnp.int32),
            pltpu.VMEM((_EPW, width), jnp.float32),
            pltpu.VMEM_SHARED((_NPAD, width), jnp.float32),
        ],
        compiler_params=pltpu.CompilerParams(use_tc_tiling_on_sc=False),
    )
    def _sc_scatter_k(rows_hbm, idx_hbm, zeros_hbm, out_hbm, idx_v, rows_v,
                      acc):
        c = lax.axis_index("c")
        s = lax.axis_index("s")
        wid = s * _NC + c
        pltpu.sync_copy(zeros_hbm, acc.at[pl.ds(s * _RPS, _RPS)])
        plsc.subcore_barrier()
        pltpu.sync_copy(idx_hbm.at[wid], idx_v)
        pltpu.sync_copy(rows_hbm.at[wid], rows_v)
        for j in range(_JCH):
            pltpu.sync_copy(
                rows_v.at[pl.ds(j * 128, 128)], acc.at[idx_v.at[j]], add=True
            )
        plsc.subcore_barrier()
        pltpu.sync_copy(acc.at[pl.ds(s * _RPS, _RPS)], out_hbm.at[c, s])

    return _sc_scatter_k


def _sc_scatter64(rows_r, idx_r, zeros):
    return _get_sc_scatter(_DIM)(rows_r, idx_r, zeros)


def _sc_scatter16(rows_r, idx_r, zeros):
    return _get_sc_scatter(16)(rows_r, idx_r, zeros)


def _tc_pre_body(x_ref, w_ref, b_ref, o_ref):
    o_ref[...] = jnp.maximum(
        jnp.dot(x_ref[...], w_ref[...], preferred_element_type=jnp.float32)
        + b_ref[...],
        0.0,
    )


def _tc_pre(x_p, lin0_w, lin0_b2):
    return pl.pallas_call(
        _tc_pre_body,
        grid=(_NPAD // 512,),
        in_specs=[
            pl.BlockSpec((512, _NF), lambda i: (i, 0)),
            pl.BlockSpec((_NF, _DIM), lambda i: (0, 0)),
            pl.BlockSpec((1, _DIM), lambda i: (0, 0)),
        ],
        out_specs=pl.BlockSpec((512, _DIM), lambda i: (i, 0)),
        out_shape=jax.ShapeDtypeStruct((_NPAD, _DIM), jnp.float32),
    )(x_p, lin0_w, lin0_b2)


_MBLK = 256


def _tc_msg_body(ea_ref, os_ref, w1_ref, b1_ref, w2_ref, nb2_ref, o_ref):
    h = jnp.maximum(
        jnp.dot(ea_ref[...], w1_ref[...], preferred_element_type=jnp.float32)
        + b1_ref[...],
        0.0,
    )
    os = os_ref[...]
    z = (os[:, :, None] * h[:, None, :]).reshape(_MBLK, _DIM * 128)
    o_ref[...] = jnp.dot(
        z, w2_ref[...], preferred_element_type=jnp.float32
    ) + jnp.dot(os, nb2_ref[...], preferred_element_type=jnp.float32)


def _tc_msg(ea_p, out_src, nn1_w, nn1_b2, w2r2, nb2):
    return pl.pallas_call(
        _tc_msg_body,
        grid=(_EPAD // _MBLK,),
        in_specs=[
            pl.BlockSpec((_MBLK, _DE), lambda i: (i, 0)),
            pl.BlockSpec((_MBLK, _DIM), lambda i: (i, 0)),
            pl.BlockSpec((_DE, 128), lambda i: (0, 0)),
            pl.BlockSpec((1, 128), lambda i: (0, 0)),
            pl.BlockSpec((_DIM * 128, _DIM), lambda i: (0, 0)),
            pl.BlockSpec((_DIM, _DIM), lambda i: (0, 0)),
        ],
        out_specs=pl.BlockSpec((_MBLK, _DIM), lambda i: (i, 0)),
        out_shape=jax.ShapeDtypeStruct((_EPAD, _DIM), jnp.float32),
    )(ea_p, out_src, nn1_w, nn1_b2, w2r2, nb2)


def _tc_node_body(
    aggA_ref, aggB_ref, cntA_ref, cntB_ref, out_ref,
    root_ref, cb_ref,
    wir_ref, wiz_ref, win_ref, whr_ref, whz_ref, whn_ref,
    bir_ref, biz_ref, bin_ref, bhr_ref, bhz_ref, bhn_ref,
    o_ref,
):
    cnt = cntA_ref[:, 0:1] + cntB_ref[:, 0:1]
    inv = 1.0 / jnp.maximum(cnt, 1.0)
    agg = (aggA_ref[...] + aggB_ref[...]) * inv
    cur = out_ref[...]
    dot = lambda a, b: jnp.dot(a, b, preferred_element_type=jnp.float32)
    m = jnp.maximum(agg + dot(cur, root_ref[...]) + cb_ref[...], 0.0)
    r = jax.nn.sigmoid(dot(m, wir_ref[...]) + bir_ref[...]
                       + dot(cur, whr_ref[...]) + bhr_ref[...])
    z = jax.nn.sigmoid(dot(m, wiz_ref[...]) + biz_ref[...]
                       + dot(cur, whz_ref[...]) + bhz_ref[...])
    n = jnp.tanh(dot(m, win_ref[...]) + bin_ref[...]
                 + r * (dot(cur, whn_ref[...]) + bhn_ref[...]))
    o_ref[...] = (1.0 - z) * n + z * cur


def _tc_node(aggA, aggB, cntA, cntB, out_cur, root, cb2, wsplits, bsplits):
    nblk = pl.BlockSpec((512, _DIM), lambda i: (i, 0))
    cblk = pl.BlockSpec((512, 16), lambda i: (i, 0))
    wblk = pl.BlockSpec((_DIM, _DIM), lambda i: (0, 0))
    bblk = pl.BlockSpec((1, _DIM), lambda i: (0, 0))
    return pl.pallas_call(
        _tc_node_body,
        grid=(_NPAD // 512,),
        in_specs=[nblk, nblk, cblk, cblk, nblk, wblk, bblk]
        + [wblk] * 6 + [bblk] * 6,
        out_specs=nblk,
        out_shape=jax.ShapeDtypeStruct((_NPAD, _DIM), jnp.float32),
    )(aggA, aggB, cntA, cntB, out_cur, root, cb2, *wsplits, *bsplits)


_NBLK = _NPAD // 512


def _tc_s2s_body(
    xn3_ref, xnT3_ref, b3_ref,
    wii_ref, wif_ref, wig_ref, wio_ref,
    whi_ref, whf_ref, whg_ref, who_ref,
    bi_ref, bf_ref, bg_ref, bo_ref,
    l1w_ref, l1b_ref,
    o_ref,
):
    dot = lambda a, b: jnp.dot(a, b, preferred_element_type=jnp.float32)
    q_star = jnp.zeros((_B, 2 * _DIM), jnp.float32)
    hc = jnp.zeros((_B, _DIM), jnp.float32)
    cc = jnp.zeros((_B, _DIM), jnp.float32)
    iota_col = lax.broadcasted_iota(jnp.int32, (_B, 1), 0)

    for _ in range(3):
        gi = jax.nn.sigmoid(dot(q_star, wii_ref[...]) + dot(hc, whi_ref[...])
                            + bi_ref[...])
        gf = jax.nn.sigmoid(dot(q_star, wif_ref[...]) + dot(hc, whf_ref[...])
                            + bf_ref[...])
        gg = jnp.tanh(dot(q_star, wig_ref[...]) + dot(hc, whg_ref[...])
                      + bg_ref[...])
        go = jax.nn.sigmoid(dot(q_star, wio_ref[...]) + dot(hc, who_ref[...])
                            + bo_ref[...])
        cc = gf * cc + gi * gg
        hc = go * jnp.tanh(cc)
        q = hc

        def _pass1(blk, emax):
            xbT = xnT3_ref[blk]
            brow = b3_ref[blk]
            maskb = brow == iota_col
            e = jnp.dot(q, xbT, preferred_element_type=jnp.float32)
            em = jnp.where(maskb, e, -jnp.inf)
            return jnp.maximum(emax, jnp.max(em, axis=1, keepdims=True))

        emax = lax.fori_loop(
            0, _NBLK, _pass1, jnp.full((_B, 1), -jnp.inf, jnp.float32)
        )
        emax = jnp.where(jnp.isfinite(emax), emax, 0.0)

        def _pass2(blk, carry):
            asum, racc = carry
            xb = xn3_ref[blk]
            xbT = xnT3_ref[blk]
            brow = b3_ref[blk]
            maskb = brow == iota_col
            e = jnp.dot(q, xbT, preferred_element_type=jnp.float32)
            a = jnp.where(maskb, jnp.exp(e - emax), 0.0)
            asum = asum + jnp.sum(a, axis=1, keepdims=True)
            racc = racc + jnp.dot(a, xb, preferred_element_type=jnp.float32)
            return asum, racc

        asum, racc = lax.fori_loop(
            0, _NBLK, _pass2,
            (jnp.zeros((_B, 1), jnp.float32), jnp.zeros((_B, _DIM), jnp.float32)),
        )
        r = racc / (asum + 1e-16)
        q_star = jnp.concatenate([q, r], axis=1)

    o_ref[...] = jnp.maximum(
        dot(q_star, l1w_ref[...]) + l1b_ref[...], 0.0
    )


def _tc_s2s(xn3, xnT3, b3, lstm_splits, lin1_w, lin1_b2):
    return pl.pallas_call(
        _tc_s2s_body,
        out_shape=jax.ShapeDtypeStruct((_B, _DIM), jnp.float32),
    )(xn3, xnT3, b3, *lstm_splits, lin1_w, lin1_b2)


def kernel(x, edge_index, edge_attr, batch, lin0_w, lin0_b, nn1_w, nn1_b,
           nn2_w, nn2_b, conv_root, conv_bias, gru_w_ih, gru_w_hh, gru_b_ih,
           gru_b_hh, lstm_w_ih, lstm_w_hh, lstm_b_ih, lstm_b_hh, lin1_w,
           lin1_b):
    f32 = jnp.float32
    x_p = jnp.pad(x, ((0, _NPAD - _N), (0, 0)))
    ea_p = jnp.pad(edge_attr, ((0, _EPAD - _E), (0, 0)))
    src_p = jnp.pad(edge_index[0], (0, _EPAD - _E))
    dst_p = jnp.pad(edge_index[1], (0, _EPAD - _E), constant_values=_N)
    batch_p = jnp.pad(batch, (0, _NPAD - _N), constant_values=_B)

    src_r = src_p.reshape(_NW, _JCH, 128)
    dst_r = dst_p.reshape(_NW, _JCH, 128)

    lin0_b2 = lin0_b.reshape(1, _DIM)
    nn1_b2 = nn1_b.reshape(1, 128)
    w2r2 = (
        nn2_w.reshape(128, _DIM, _DIM).transpose(1, 0, 2).reshape(128 * _DIM, _DIM)
    )
    nb2 = nn2_b.reshape(_DIM, _DIM)
    cb2 = conv_bias.reshape(1, _DIM)
    gw = [gru_w_ih[:, k * _DIM:(k + 1) * _DIM] for k in range(3)] + \
         [gru_w_hh[:, k * _DIM:(k + 1) * _DIM] for k in range(3)]
    gb = [gru_b_ih[k * _DIM:(k + 1) * _DIM].reshape(1, _DIM) for k in range(3)] + \
         [gru_b_hh[k * _DIM:(k + 1) * _DIM].reshape(1, _DIM) for k in range(3)]
    lw = [lstm_w_ih[:, k * _DIM:(k + 1) * _DIM] for k in range(4)] + \
         [lstm_w_hh[:, k * _DIM:(k + 1) * _DIM] for k in range(4)]
    lb = [(lstm_b_ih[k * _DIM:(k + 1) * _DIM]
           + lstm_b_hh[k * _DIM:(k + 1) * _DIM]).reshape(1, _DIM)
          for k in range(4)]
    lin1_b2 = lin1_b.reshape(1, _DIM)

    ones16 = jnp.ones((_EPW, 16), f32)
    ones16_r = jnp.broadcast_to(ones16, (_NW, _EPW, 16))
    zeros16 = jnp.zeros((_RPS, 16), f32)
    zeros64 = jnp.zeros((_RPS, _DIM), f32)

    cnt_parts = _sc_scatter16(ones16_r, dst_r, zeros16)
    cnt_parts = cnt_parts.reshape(_NC, _NPAD, 16)
    cntA, cntB = cnt_parts[0], cnt_parts[1]

    out = _tc_pre(x_p, lin0_w, lin0_b2)

    for _ in range(3):
        out_src = _sc_gather(out, src_r).reshape(_EPAD, _DIM)
        msg = _tc_msg(ea_p, out_src, nn1_w, nn1_b2, w2r2, nb2)
        parts = _sc_scatter64(msg.reshape(_NW, _EPW, _DIM), dst_r, zeros64)
        parts = parts.reshape(_NC, _NPAD, _DIM)
        out = _tc_node(parts[0], parts[1], cntA, cntB, out, conv_root, cb2,
                       gw, gb)

    xn3 = out.reshape(_NBLK, 512, _DIM)
    xnT3 = out.T.reshape(_DIM, _NBLK, 512).transpose(1, 0, 2)
    b3 = batch_p.reshape(_NBLK, 1, 512)
    return _tc_s2s(xn3, xnT3, b3, lw + lb, lin1_w, lin1_b2)


# speedup vs baseline: 2.5985x; 1.5339x over previous
import functools

import jax
import jax.numpy as jnp
from jax import lax
from jax.experimental import pallas as pl
from jax.experimental.pallas import tpu as pltpu
from jax.experimental.pallas import tpu_sc as plsc

_N = 10000
_E = 20000
_B = 512
_DIM = 64
_NF = 11
_DE = 5

_NPAD = 10240
_EPAD = 20480
_NC = 2
_NS = 16
_NW = _NC * _NS
_EPW = _EPAD // _NW
_JCH = _EPW // 128
_RPS = _NPAD // _NS


@functools.cache
def _get_sc_gather():
    mesh = plsc.VectorSubcoreMesh(core_axis_name="c", subcore_axis_name="s")

    @functools.partial(
        pl.kernel,
        out_type=jax.ShapeDtypeStruct((_NW, _EPW, _DIM), jnp.float32),
        mesh=mesh,
        scratch_types=[
            pltpu.VMEM((_JCH, 128), jnp.int32),
            pltpu.VMEM((_EPW, _DIM), jnp.float32),
            pltpu.SemaphoreType.DMA,
        ],
        compiler_params=pltpu.CompilerParams(use_tc_tiling_on_sc=False),
    )
    def _sc_gather_k(table_hbm, idx_hbm, out_hbm, idx_v, rows_v, sem):
        c = lax.axis_index("c")
        s = lax.axis_index("s")
        wid = s * _NC + c
        pltpu.sync_copy(idx_hbm.at[wid], idx_v)
        descs = []
        for j in range(_JCH):
            descs.append(
                pltpu.async_copy(
                    table_hbm.at[idx_v.at[j]], rows_v.at[pl.ds(j * 128, 128)],
                    sem,
                )
            )
        for d in descs:
            d.wait()
        pltpu.sync_copy(rows_v, out_hbm.at[wid])

    return _sc_gather_k


def _sc_gather(table, idx_r):
    return _get_sc_gather()(table, idx_r)


@functools.cache
def _get_sc_scatter(width):
    mesh = plsc.VectorSubcoreMesh(core_axis_name="c", subcore_axis_name="s")

    @functools.partial(
        pl.kernel,
        out_type=jax.ShapeDtypeStruct((_NC, _NS, _RPS, width), jnp.float32),
        mesh=mesh,
        scratch_types=[
            pltpu.VMEM((_JCH, 128), jnp.int32),
            pltpu.VMEM((_EPW, width), jnp.float32),
            pltpu.VMEM_SHARED((_NPAD, width), jnp.float32),
        ],
        compiler_params=pltpu.CompilerParams(use_tc_tiling_on_sc=False),
    )
    def _sc_scatter_k(rows_hbm, idx_hbm, zeros_hbm, out_hbm, idx_v, rows_v,
                      acc):
        c = lax.axis_index("c")
        s = lax.axis_index("s")
        wid = s * _NC + c
        pltpu.sync_copy(zeros_hbm, acc.at[pl.ds(s * _RPS, _RPS)])
        plsc.subcore_barrier()
        pltpu.sync_copy(idx_hbm.at[wid], idx_v)
        pltpu.sync_copy(rows_hbm.at[wid], rows_v)
        for j in range(_JCH):
            pltpu.sync_copy(
                rows_v.at[pl.ds(j * 128, 128)], acc.at[idx_v.at[j]], add=True
            )
        plsc.subcore_barrier()
        pltpu.sync_copy(acc.at[pl.ds(s * _RPS, _RPS)], out_hbm.at[c, s])

    return _sc_scatter_k


def _sc_scatter64(rows_r, idx_r, zeros):
    return _get_sc_scatter(_DIM)(rows_r, idx_r, zeros)


def _sc_scatter16(rows_r, idx_r, zeros):
    return _get_sc_scatter(16)(rows_r, idx_r, zeros)


def _tc_pre_body(x_ref, w_ref, b_ref, o_ref):
    o_ref[...] = jnp.maximum(
        jnp.dot(x_ref[...], w_ref[...], preferred_element_type=jnp.float32)
        + b_ref[...],
        0.0,
    )


def _tc_pre(x_p, lin0_w, lin0_b2):
    return pl.pallas_call(
        _tc_pre_body,
        grid=(_NPAD // 512,),
        in_specs=[
            pl.BlockSpec((512, _NF), lambda i: (i, 0)),
            pl.BlockSpec((_NF, _DIM), lambda i: (0, 0)),
            pl.BlockSpec((1, _DIM), lambda i: (0, 0)),
        ],
        out_specs=pl.BlockSpec((512, _DIM), lambda i: (i, 0)),
        out_shape=jax.ShapeDtypeStruct((_NPAD, _DIM), jnp.float32),
    )(x_p, lin0_w, lin0_b2)


_MBLK = 512


def _tc_msg_body(ea_ref, os_ref, w1_ref, b1_ref, w2_ref, nb2_ref, p2_ref,
                 o_ref):
    h = jnp.maximum(
        jnp.dot(ea_ref[...], w1_ref[...], preferred_element_type=jnp.float32)
        + b1_ref[...],
        0.0,
    )
    h_b = h.astype(jnp.bfloat16)
    g = jnp.dot(h_b, w2_ref[...], preferred_element_type=jnp.float32)
    os = os_ref[...]
    os_exp = jnp.dot(os.astype(jnp.bfloat16), p2_ref[...],
                     preferred_element_type=jnp.float32)
    m = os_exp * g
    w = _DIM * _DIM
    while w > _DIM:
        w //= 2
        m = m[:, :w] + m[:, w:2 * w]
    o_ref[...] = m + jnp.dot(
        os, nb2_ref[...], preferred_element_type=jnp.float32
    )


def _tc_msg(ea_p, out_src, nn1_w, nn1_b2, w2r2, nb2, p2):
    return pl.pallas_call(
        _tc_msg_body,
        grid=(_EPAD // _MBLK,),
        in_specs=[
            pl.BlockSpec((_MBLK, _DE), lambda i: (i, 0)),
            pl.BlockSpec((_MBLK, _DIM), lambda i: (i, 0)),
            pl.BlockSpec((_DE, 128), lambda i: (0, 0)),
            pl.BlockSpec((1, 128), lambda i: (0, 0)),
            pl.BlockSpec((128, _DIM * _DIM), lambda i: (0, 0)),
            pl.BlockSpec((_DIM, _DIM), lambda i: (0, 0)),
            pl.BlockSpec((_DIM, _DIM * _DIM), lambda i: (0, 0)),
        ],
        out_specs=pl.BlockSpec((_MBLK, _DIM), lambda i: (i, 0)),
        out_shape=jax.ShapeDtypeStruct((_EPAD, _DIM), jnp.float32),
    )(ea_p, out_src, nn1_w, nn1_b2, w2r2, nb2, p2)


def _tc_node_body(
    aggA_ref, aggB_ref, cntA_ref, cntB_ref, out_ref,
    root_ref, cb_ref,
    wir_ref, wiz_ref, win_ref, whr_ref, whz_ref, whn_ref,
    bir_ref, biz_ref, bin_ref, bhr_ref, bhz_ref, bhn_ref,
    o_ref,
):
    cnt = cntA_ref[:, 0:1] + cntB_ref[:, 0:1]
    inv = 1.0 / jnp.maximum(cnt, 1.0)
    agg = (aggA_ref[...] + aggB_ref[...]) * inv
    cur = out_ref[...]
    dot = lambda a, b: jnp.dot(a, b, preferred_element_type=jnp.float32)
    m = jnp.maximum(agg + dot(cur, root_ref[...]) + cb_ref[...], 0.0)
    r = jax.nn.sigmoid(dot(m, wir_ref[...]) + bir_ref[...]
                       + dot(cur, whr_ref[...]) + bhr_ref[...])
    z = jax.nn.sigmoid(dot(m, wiz_ref[...]) + biz_ref[...]
                       + dot(cur, whz_ref[...]) + bhz_ref[...])
    n = jnp.tanh(dot(m, win_ref[...]) + bin_ref[...]
                 + r * (dot(cur, whn_ref[...]) + bhn_ref[...]))
    o_ref[...] = (1.0 - z) * n + z * cur


def _tc_node(aggA, aggB, cntA, cntB, out_cur, root, cb2, wsplits, bsplits):
    nblk = pl.BlockSpec((512, _DIM), lambda i: (i, 0))
    cblk = pl.BlockSpec((512, 16), lambda i: (i, 0))
    wblk = pl.BlockSpec((_DIM, _DIM), lambda i: (0, 0))
    bblk = pl.BlockSpec((1, _DIM), lambda i: (0, 0))
    return pl.pallas_call(
        _tc_node_body,
        grid=(_NPAD // 512,),
        in_specs=[nblk, nblk, cblk, cblk, nblk, wblk, bblk]
        + [wblk] * 6 + [bblk] * 6,
        out_specs=nblk,
        out_shape=jax.ShapeDtypeStruct((_NPAD, _DIM), jnp.float32),
    )(aggA, aggB, cntA, cntB, out_cur, root, cb2, *wsplits, *bsplits)


_NBLK = _NPAD // 512


def _tc_s2s_body(
    xn3_ref, xnT3_ref, b3_ref,
    wii_ref, wif_ref, wig_ref, wio_ref,
    whi_ref, whf_ref, whg_ref, who_ref,
    bi_ref, bf_ref, bg_ref, bo_ref,
    l1w_ref, l1b_ref,
    o_ref,
):
    dot = lambda a, b: jnp.dot(a, b, preferred_element_type=jnp.float32)
    q_star = jnp.zeros((_B, 2 * _DIM), jnp.float32)
    hc = jnp.zeros((_B, _DIM), jnp.float32)
    cc = jnp.zeros((_B, _DIM), jnp.float32)
    iota_col = lax.broadcasted_iota(jnp.int32, (_B, 1), 0)

    for _ in range(3):
        gi = jax.nn.sigmoid(dot(q_star, wii_ref[...]) + dot(hc, whi_ref[...])
                            + bi_ref[...])
        gf = jax.nn.sigmoid(dot(q_star, wif_ref[...]) + dot(hc, whf_ref[...])
                            + bf_ref[...])
        gg = jnp.tanh(dot(q_star, wig_ref[...]) + dot(hc, whg_ref[...])
                      + bg_ref[...])
        go = jax.nn.sigmoid(dot(q_star, wio_ref[...]) + dot(hc, who_ref[...])
                            + bo_ref[...])
        cc = gf * cc + gi * gg
        hc = go * jnp.tanh(cc)
        q = hc

        def _pass1(blk, emax):
            xbT = xnT3_ref[blk]
            brow = b3_ref[blk]
            maskb = brow == iota_col
            e = jnp.dot(q, xbT, preferred_element_type=jnp.float32)
            em = jnp.where(maskb, e, -jnp.inf)
            return jnp.maximum(emax, jnp.max(em, axis=1, keepdims=True))

        emax = lax.fori_loop(
            0, _NBLK, _pass1, jnp.full((_B, 1), -jnp.inf, jnp.float32)
        )
        emax = jnp.where(jnp.isfinite(emax), emax, 0.0)

        def _pass2(blk, carry):
            asum, racc = carry
            xb = xn3_ref[blk]
            xbT = xnT3_ref[blk]
            brow = b3_ref[blk]
            maskb = brow == iota_col
            e = jnp.dot(q, xbT, preferred_element_type=jnp.float32)
            a = jnp.where(maskb, jnp.exp(e - emax), 0.0)
            asum = asum + jnp.sum(a, axis=1, keepdims=True)
            racc = racc + jnp.dot(a, xb, preferred_element_type=jnp.float32)
            return asum, racc

        asum, racc = lax.fori_loop(
            0, _NBLK, _pass2,
            (jnp.zeros((_B, 1), jnp.float32), jnp.zeros((_B, _DIM), jnp.float32)),
        )
        r = racc / (asum + 1e-16)
        q_star = jnp.concatenate([q, r], axis=1)

    o_ref[...] = jnp.maximum(
        dot(q_star, l1w_ref[...]) + l1b_ref[...], 0.0
    )


def _tc_s2s(xn3, xnT3, b3, lstm_splits, lin1_w, lin1_b2):
    return pl.pallas_call(
        _tc_s2s_body,
        out_shape=jax.ShapeDtypeStruct((_B, _DIM), jnp.float32),
    )(xn3, xnT3, b3, *lstm_splits, lin1_w, lin1_b2)


def kernel(x, edge_index, edge_attr, batch, lin0_w, lin0_b, nn1_w, nn1_b,
           nn2_w, nn2_b, conv_root, conv_bias, gru_w_ih, gru_w_hh, gru_b_ih,
           gru_b_hh, lstm_w_ih, lstm_w_hh, lstm_b_ih, lstm_b_hh, lin1_w,
           lin1_b):
    f32 = jnp.float32
    x_p = jnp.pad(x, ((0, _NPAD - _N), (0, 0)))
    ea_p = jnp.pad(edge_attr, ((0, _EPAD - _E), (0, 0)))
    src_p = jnp.pad(edge_index[0], (0, _EPAD - _E))
    dst_p = jnp.pad(edge_index[1], (0, _EPAD - _E), constant_values=_N)
    batch_p = jnp.pad(batch, (0, _NPAD - _N), constant_values=_B)

    src_r = src_p.reshape(_NW, _JCH, 128)
    dst_r = dst_p.reshape(_NW, _JCH, 128)

    lin0_b2 = lin0_b.reshape(1, _DIM)
    nn1_b2 = nn1_b.reshape(1, 128)
    w2r2 = nn2_w.astype(jnp.bfloat16)
    p2 = jnp.repeat(jnp.eye(_DIM, dtype=jnp.bfloat16), _DIM, axis=1)
    nb2 = nn2_b.reshape(_DIM, _DIM)
    cb2 = conv_bias.reshape(1, _DIM)
    gw = [gru_w_ih[:, k * _DIM:(k + 1) * _DIM] for k in range(3)] + \
         [gru_w_hh[:, k * _DIM:(k + 1) * _DIM] for k in range(3)]
    gb = [gru_b_ih[k * _DIM:(k + 1) * _DIM].reshape(1, _DIM) for k in range(3)] + \
         [gru_b_hh[k * _DIM:(k + 1) * _DIM].reshape(1, _DIM) for k in range(3)]
    lw = [lstm_w_ih[:, k * _DIM:(k + 1) * _DIM] for k in range(4)] + \
         [lstm_w_hh[:, k * _DIM:(k + 1) * _DIM] for k in range(4)]
    lb = [(lstm_b_ih[k * _DIM:(k + 1) * _DIM]
           + lstm_b_hh[k * _DIM:(k + 1) * _DIM]).reshape(1, _DIM)
          for k in range(4)]
    lin1_b2 = lin1_b.reshape(1, _DIM)

    ones16 = jnp.ones((_EPW, 16), f32)
    ones16_r = jnp.broadcast_to(ones16, (_NW, _EPW, 16))
    zeros16 = jnp.zeros((_RPS, 16), f32)
    zeros64 = jnp.zeros((_RPS, _DIM), f32)

    cnt_parts = _sc_scatter16(ones16_r, dst_r, zeros16)
    cnt_parts = cnt_parts.reshape(_NC, _NPAD, 16)
    cntA, cntB = cnt_parts[0], cnt_parts[1]

    out = _tc_pre(x_p, lin0_w, lin0_b2)

    for _ in range(3):
        out_src = _sc_gather(out, src_r).reshape(_EPAD, _DIM)
        msg = _tc_msg(ea_p, out_src, nn1_w, nn1_b2, w2r2, nb2, p2)
        parts = _sc_scatter64(msg.reshape(_NW, _EPW, _DIM), dst_r, zeros64)
        parts = parts.reshape(_NC, _NPAD, _DIM)
        out = _tc_node(parts[0], parts[1], cntA, cntB, out, conv_root, cb2,
                       gw, gb)

    xn3 = out.reshape(_NBLK, 512, _DIM)
    xnT3 = out.T.reshape(_DIM, _NBLK, 512).transpose(1, 0, 2)
    b3 = batch_p.reshape(_NBLK, 1, 512)
    return _tc_s2s(xn3, xnT3, b3, lw + lb, lin1_w, lin1_b2)
